# baseline (speedup 1.0000x reference)
import jax
import jax.numpy as jnp
from jax.experimental import pallas as pl
from jax.experimental.pallas import tpu as pltpu

_K = 10
_ROWS = 256
_G = 8
_NG = _ROWS // _G
_H = 16


def _cumsum_lanes(x):
    n = x.shape[-1]
    shift = 1
    while shift < n:
        shifted = jnp.concatenate(
            [jnp.zeros(x.shape[:-1] + (shift,), x.dtype), x[..., :-shift]], axis=-1)
        x = x + shifted
        shift *= 2
    return x


def _tc_kernel(embt_ref, emb_ref, alpha_ref, phys_ref, out_ref, a_ref):
    embt = embt_ref[0]
    emb_rows = emb_ref[0].reshape(_ROWS, _H)
    c = jax.nn.sigmoid(alpha_ref[0, 0])
    n = embt.shape[1]

    a2 = jax.lax.dot_general(emb_rows, embt, (((1,), (0,)), ((), ())),
                             preferred_element_type=jnp.float32)
    a_ref[...] = jnp.maximum(a2, 0.0).reshape(_NG, _G, n)

    def group(j, bad):
        ag = a_ref[j]
        cur = None
        rmax = None
        for k in range(_K):
            masked = ag if k == 0 else jnp.where(ag < cur, ag, -1.0)
            cur = jnp.max(masked, axis=1, keepdims=True)
            if k == 0:
                rmax = cur
        selw = jnp.where(ag >= cur, 1.0, 0.0)
        cnt = jnp.sum(selw, axis=1, keepdims=True)
        e = selw * jnp.exp(ag - rmax)
        z = jnp.sum(e, axis=1, keepdims=True)
        pg = phys_ref[j]
        ps = jnp.sum(pg, axis=1, keepdims=True) + 1e-8
        out_ref[0, j] = (c / ps) * pg + ((1.0 - c) / z) * e
        return jnp.maximum(bad, jnp.max(jnp.abs(cnt - float(_K))))

    bad = jax.lax.fori_loop(0, _NG, group, jnp.float32(0.0))

    @pl.when(bad > 0.0)
    def _slow():
        a = a_ref[...].reshape(_ROWS, n)
        r = _ROWS
        cur = jnp.full((r, 1), jnp.inf, dtype=jnp.float32)
        thr = jnp.zeros((r, 1), dtype=jnp.float32)
        row_max = jnp.zeros((r, 1), dtype=jnp.float32)
        for k in range(_K):
            lt = a < cur
            n_ge = float(n) - jnp.sum(jnp.where(lt, 1.0, 0.0), axis=1,
                                      keepdims=True)
            d = jnp.max(jnp.where(lt, a, -1.0), axis=1, keepdims=True)
            take = n_ge < float(_K)
            thr = jnp.where(take, d, thr)
            if k == 0:
                row_max = d
            cur = d
        mask_gt = a > thr
        mask_eq = a == thr
        cnt_gt = jnp.sum(jnp.where(mask_gt, 1.0, 0.0), axis=1, keepdims=True)
        extra = float(_K) - cnt_gt
        rank = _cumsum_lanes(jnp.where(mask_eq, 1.0, 0.0))
        keep = jnp.logical_and(mask_eq, rank <= extra)
        sel = jnp.where(jnp.logical_or(mask_gt, keep), 1.0, 0.0)
        e = sel * jnp.exp(a - row_max)
        z = jnp.sum(e, axis=1, keepdims=True)
        phys = phys_ref[...].reshape(r, n)
        psum = jnp.sum(phys, axis=1, keepdims=True) + 1e-8
        res = (c / psum) * phys + ((1.0 - c) / z) * e
        out_ref[...] = res.reshape(1, _NG, _G, n)


def kernel(x, A_physical, W, b, alpha):
    bsz, _, n, _ = x.shape
    state = x[:, -1, :, :]
    emb = jnp.tanh(state @ W + b)
    embt = jnp.swapaxes(emb, 1, 2)
    emb4 = emb.reshape(bsz, n // _G, _G, _H)
    phys3 = A_physical.reshape(n // _G, _G, n)
    alpha2 = jnp.asarray(alpha, jnp.float32).reshape(1, 1)
    grid = (n // _ROWS, bsz)
    out = pl.pallas_call(
        _tc_kernel,
        grid=grid,
        in_specs=[
            pl.BlockSpec((1, _H, n), lambda i, bb: (bb, 0, 0)),
            pl.BlockSpec((1, _NG, _G, _H), lambda i, bb: (bb, i, 0, 0)),
            pl.BlockSpec((1, 1), lambda i, bb: (0, 0)),
            pl.BlockSpec((_NG, _G, n), lambda i, bb: (i, 0, 0)),
        ],
        out_specs=pl.BlockSpec((1, _NG, _G, n), lambda i, bb: (bb, i, 0, 0)),
        out_shape=jax.ShapeDtypeStruct((bsz, n // _G, _G, n), jnp.float32),
        scratch_shapes=[pltpu.VMEM((_NG, _G, n), jnp.float32)],
    )(embt, emb4, alpha2, phys3)
    return out.reshape(bsz, n, n)


# speedup vs baseline: 2.7643x; 2.7643x over previous
import functools

import jax
import jax.numpy as jnp
from jax.experimental import pallas as pl

_K = 10
_ROWS = 256
_H = 16


def _cumsum_lanes(x):
    n = x.shape[-1]
    shift = 1
    while shift < n:
        shifted = jnp.concatenate(
            [jnp.zeros(x.shape[:-1] + (shift,), x.dtype), x[..., :-shift]], axis=-1)
        x = x + shifted
        shift *= 2
    return x


def _common(embt_ref, emb_rows_ref, alpha_ref):
    embt = embt_ref[0]
    emb_rows = emb_rows_ref[0]
    c = jax.nn.sigmoid(alpha_ref[0, 0])
    a = jax.lax.dot_general(emb_rows, embt, (((1,), (0,)), ((), ())),
                            preferred_element_type=jnp.float32)
    a = jnp.maximum(a, 0.0)
    n = a.shape[1]
    ones = jnp.ones((n, 1), dtype=jnp.float32)

    def rowcount(mask):
        return jax.lax.dot_general(jnp.where(mask, 1.0, 0.0), ones,
                                   (((1,), (0,)), ((), ())),
                                   preferred_element_type=jnp.float32)

    r = a.shape[0]
    cur = jnp.full((r, 1), jnp.inf, dtype=jnp.float32)
    thr = jnp.zeros((r, 1), dtype=jnp.float32)
    row_max = jnp.zeros((r, 1), dtype=jnp.float32)
    for k in range(_K):
        lt = a < cur
        n_ge = float(n) - rowcount(lt)
        d = jnp.max(jnp.where(lt, a, -1.0), axis=1, keepdims=True)
        take = n_ge < float(_K)
        thr = jnp.where(take, d, thr)
        if k == 0:
            row_max = d
        cur = d
    return a, c, thr, row_max, rowcount


def _emit(out_ref, phys_ref, a, c, row_max, sel):
    e = sel * jnp.exp(a - row_max)
    z = jnp.sum(e, axis=1, keepdims=True)
    phys = phys_ref[...]
    psum = jnp.sum(phys, axis=1, keepdims=True) + 1e-8
    out_ref[0, :, :] = (c / psum) * phys + ((1.0 - c) / z) * e


def _fast_kernel(embt_ref, emb_rows_ref, alpha_ref, phys_ref, out_ref,
                 flag_ref):
    a, c, thr, row_max, rowcount = _common(embt_ref, emb_rows_ref, alpha_ref)
    cnt_gt = rowcount(a > thr)
    cnt_eq = rowcount(a == thr)
    excess = jnp.max(cnt_eq - (float(_K) - cnt_gt))
    flag_ref[0, :, :] = jnp.full(flag_ref.shape[1:], excess, jnp.float32)
    _emit(out_ref, phys_ref, a, c, row_max,
          jnp.where(a >= thr, 1.0, 0.0))


def _slow_kernel(embt_ref, emb_rows_ref, alpha_ref, phys_ref, out_ref):
    a, c, thr, row_max, rowcount = _common(embt_ref, emb_rows_ref, alpha_ref)
    mask_gt = a > thr
    mask_eq = a == thr
    extra = float(_K) - rowcount(mask_gt)
    rank = _cumsum_lanes(jnp.where(mask_eq, 1.0, 0.0))
    keep = jnp.logical_and(mask_eq, rank <= extra)
    _emit(out_ref, phys_ref, a, c, row_max,
          jnp.where(jnp.logical_or(mask_gt, keep), 1.0, 0.0))


def kernel(x, A_physical, W, b, alpha):
    bsz, _, n, _ = x.shape
    state = x[:, -1, :, :]
    emb = jnp.tanh(state @ W + b)
    embt = jnp.swapaxes(emb, 1, 2)
    alpha2 = jnp.asarray(alpha, jnp.float32).reshape(1, 1)
    ni = n // _ROWS
    grid = (ni, bsz)
    in_specs = [
        pl.BlockSpec((1, _H, n), lambda i, bb: (bb, 0, 0)),
        pl.BlockSpec((1, _ROWS, _H), lambda i, bb: (bb, i, 0)),
        pl.BlockSpec((1, 1), lambda i, bb: (0, 0)),
        pl.BlockSpec((_ROWS, n), lambda i, bb: (i, 0)),
    ]
    out_spec = pl.BlockSpec((1, _ROWS, n), lambda i, bb: (bb, i, 0))
    out_sds = jax.ShapeDtypeStruct((bsz, n, n), jnp.float32)
    args = (embt, emb, alpha2, A_physical)

    out_fast, flags = pl.pallas_call(
        _fast_kernel,
        grid=grid,
        in_specs=in_specs,
        out_specs=(out_spec,
                   pl.BlockSpec((1, 8, 128),
                                lambda i, bb: (i * bsz + bb, 0, 0))),
        out_shape=(out_sds,
                   jax.ShapeDtypeStruct((ni * bsz, 8, 128), jnp.float32)),
    )(*args)

    slow = functools.partial(
        pl.pallas_call(
            _slow_kernel,
            grid=grid,
            in_specs=in_specs,
            out_specs=out_spec,
            out_shape=out_sds,
        ), *args)

    return jax.lax.cond(jnp.max(flags[:, 0, 0]) > 0.5,
                        slow, lambda: out_fast)


# speedup vs baseline: 6.0163x; 2.1764x over previous
import jax
import jax.numpy as jnp
from jax.experimental import pallas as pl


def _cumsum_lanes(x):
    n = x.shape[-1]
    shift = 1
    while shift < n:
        shifted = jnp.concatenate(
            [jnp.zeros(x.shape[:-1] + (shift,), x.dtype), x[..., :-shift]], axis=-1)
        x = x + shifted
        shift *= 2
    return x


_K = 10
_ROWS = 256
_H = 16
_C = 128


def _tc_kernel(embt_ref, emb_rows_ref, alpha_ref, phys_ref, out_ref):
    embt = embt_ref[0]
    emb_rows = emb_rows_ref[0]
    c = jax.nn.sigmoid(alpha_ref[0, 0])

    a = jax.lax.dot_general(emb_rows, embt, (((1,), (0,)), ((), ())),
                            preferred_element_type=jnp.float32)
    a = jnp.maximum(a, 0.0)
    r, n = a.shape
    ones = jnp.ones((n, 1), dtype=jnp.float32)

    def rowcount(mask):
        return jax.lax.dot_general(jnp.where(mask, 1.0, 0.0), ones,
                                   (((1,), (0,)), ((), ())),
                                   preferred_element_type=jnp.float32)

    cur = jnp.full((r, 1), jnp.inf, dtype=jnp.float32)
    thr = jnp.zeros((r, 1), dtype=jnp.float32)
    row_max = jnp.zeros((r, 1), dtype=jnp.float32)
    for k in range(_K):
        lt = a < cur
        n_ge = float(n) - rowcount(lt)
        d = jnp.max(jnp.where(lt, a, -1.0), axis=1, keepdims=True)
        take = n_ge < float(_K)
        thr = jnp.where(take, d, thr)
        if k == 0:
            row_max = d
        cur = d

    extra = float(_K) - rowcount(a > thr)

    eqf = jnp.where(a == thr, 1.0, 0.0)
    li = jax.lax.broadcasted_iota(jnp.int32, (_C, _C), 0)
    lj = jax.lax.broadcasted_iota(jnp.int32, (_C, _C), 1)
    tri = jnp.where(li <= lj, 1.0, 0.0).astype(jnp.float32)
    nc = n // _C
    parts = [jax.lax.dot_general(eqf[:, j * _C:(j + 1) * _C], tri,
                                 (((1,), (0,)), ((), ())),
                                 preferred_element_type=jnp.float32)
             for j in range(nc)]
    tot = jnp.concatenate([p[:, _C - 1:_C] for p in parts], axis=1)
    offs = _cumsum_lanes(tot) - tot
    rank = jnp.concatenate(
        [parts[j] + offs[:, j:j + 1] for j in range(nc)], axis=1)

    sel = jnp.where(
        jnp.logical_or(a > thr,
                       jnp.logical_and(a == thr, rank <= extra)), 1.0, 0.0)

    e = sel * jnp.exp(a - row_max)
    z = jnp.sum(e, axis=1, keepdims=True)
    phys = phys_ref[...]
    psum = jnp.sum(phys, axis=1, keepdims=True) + 1e-8
    out_ref[0, :, :] = (c / psum) * phys + ((1.0 - c) / z) * e


def kernel(x, A_physical, W, b, alpha):
    bsz, _, n, _ = x.shape
    state = x[:, -1, :, :]
    emb = jnp.tanh(state @ W + b)
    embt = jnp.swapaxes(emb, 1, 2)
    alpha2 = jnp.asarray(alpha, jnp.float32).reshape(1, 1)
    grid = (n // _ROWS, bsz)
    return pl.pallas_call(
        _tc_kernel,
        grid=grid,
        in_specs=[
            pl.BlockSpec((1, _H, n), lambda i, bb: (bb, 0, 0)),
            pl.BlockSpec((1, _ROWS, _H), lambda i, bb: (bb, i, 0)),
            pl.BlockSpec((1, 1), lambda i, bb: (0, 0)),
            pl.BlockSpec((_ROWS, n), lambda i, bb: (i, 0)),
        ],
        out_specs=pl.BlockSpec((1, _ROWS, n), lambda i, bb: (bb, i, 0)),
        out_shape=jax.ShapeDtypeStruct((bsz, n, n), jnp.float32),
    )(embt, emb, alpha2, A_physical)
